# baseline (speedup 1.0000x reference)
import numpy as np
import jax
import jax.numpy as jnp
from jax import lax
from jax.experimental import pallas as pl
from jax.experimental.pallas import tpu as pltpu
from jax.experimental.pallas import tpu_sc as plsc

_VOCAB = 1000000
_EMBED = 64
_MAXLEN = 100
_BATCH = 4096
_SCALE = 8.0

_ROWS = _BATCH * _MAXLEN
_SEQ_PER_CHUNK = 4
_C = _SEQ_PER_CHUNK * _MAXLEN
_LANES = 16
_DSL = _EMBED // _LANES


def _pos_encoding():
    p, i = np.meshgrid(np.arange(_MAXLEN), 2 * np.arange(_EMBED // 2))
    pos = np.empty((_MAXLEN, _EMBED))
    pos[:, ::2] = np.sin(p / 10000 ** (i / _EMBED)).T
    pos[:, 1::2] = np.cos(p / 10000 ** (i / _EMBED)).T
    return pos.astype(np.float32)


def _make_body(nw):
    per_w = _ROWS // nw
    nchunk = per_w // _C

    def body(x_hbm, pos_hbm, table_hbm, out_hbm, idx_v, rows_v, pos_v, gsem):
        cid = lax.axis_index("c")
        sid = lax.axis_index("s")
        wid = sid * 2 + cid
        pltpu.sync_copy(pos_hbm, pos_v)

        def chunk(ci, carry):
            goff = wid * per_w + ci * _C
            pltpu.sync_copy(x_hbm.at[wid * nchunk + ci], idx_v)
            copies = [
                pltpu.async_copy(
                    table_hbm.at[idx_v.at[j]],
                    rows_v.at[pl.ds(j * _MAXLEN, _MAXLEN)],
                    gsem,
                )
                for j in range(_SEQ_PER_CHUNK)
            ]
            for c in copies:
                c.wait()

            def rowfn(r, rcarry):
                for d in range(_DSL):
                    sl = pl.ds(d * _LANES, _LANES)
                    rows_v[r, sl] = rows_v[r, sl] * _SCALE + pos_v[r, sl]
                return rcarry

            lax.fori_loop(0, _C, rowfn, 0)
            pltpu.sync_copy(rows_v, out_hbm.at[pl.ds(goff, _C)])
            return carry

        lax.fori_loop(0, nchunk, chunk, 0)

    return body


def kernel(x, table):
    info = plsc.get_sparse_core_info()
    nw = info.num_cores * info.num_subcores
    pos_rep = jnp.asarray(np.tile(_pos_encoding(), (_SEQ_PER_CHUNK, 1)))
    nchunk = (_ROWS // nw) // _C
    x32 = x.reshape(nw * nchunk, _SEQ_PER_CHUNK, _MAXLEN).astype(jnp.int32)

    mesh = plsc.VectorSubcoreMesh(core_axis_name="c", subcore_axis_name="s")
    kfn = pl.kernel(
        _make_body(nw),
        mesh=mesh,
        compiler_params=pltpu.CompilerParams(use_tc_tiling_on_sc=False),
        out_type=jax.ShapeDtypeStruct((_ROWS, _EMBED), jnp.float32),
        scratch_types=[
            pltpu.VMEM((_SEQ_PER_CHUNK, _MAXLEN), jnp.int32),
            pltpu.VMEM((_C, _EMBED), jnp.float32),
            pltpu.VMEM((_C, _EMBED), jnp.float32),
            pltpu.SemaphoreType.DMA,
        ],
    )
    out = kfn(x32, pos_rep, table)
    return out.reshape(_BATCH, _MAXLEN, _EMBED)


# speedup vs baseline: 1.1050x; 1.1050x over previous
import numpy as np
import jax
import jax.numpy as jnp
from jax import lax
from jax.experimental import pallas as pl
from jax.experimental.pallas import tpu as pltpu
from jax.experimental.pallas import tpu_sc as plsc

_VOCAB = 1000000
_EMBED = 64
_MAXLEN = 100
_BATCH = 4096
_SCALE = 8.0

_ROWS = _BATCH * _MAXLEN
_SEQ_PER_CHUNK = 8
_C = _SEQ_PER_CHUNK * _MAXLEN
_LANES = 16
_DSL = _EMBED // _LANES


def _pos_encoding():
    p, i = np.meshgrid(np.arange(_MAXLEN), 2 * np.arange(_EMBED // 2))
    pos = np.empty((_MAXLEN, _EMBED))
    pos[:, ::2] = np.sin(p / 10000 ** (i / _EMBED)).T
    pos[:, 1::2] = np.cos(p / 10000 ** (i / _EMBED)).T
    return pos.astype(np.float32)


def _make_body(nw, nchunk):
    per_w = nchunk * _C

    def body(x_hbm, pos_hbm, table_hbm, out_hbm,
             idx0, idx1, rows0, rows1, pos_v, g0, g1, o0, o1):
        cid = lax.axis_index("c")
        sid = lax.axis_index("s")
        wid = sid * 2 + cid
        pltpu.sync_copy(pos_hbm, pos_v)

        idx = [idx0, idx1]
        rows = [rows0, rows1]
        gsem = [g0, g1]
        osem = [o0, o1]
        out_dma = [None, None]

        def stage(ci, b):
            pltpu.sync_copy(x_hbm.at[wid * nchunk + ci], idx[b])
            return [
                pltpu.async_copy(
                    table_hbm.at[idx[b].at[j]],
                    rows[b].at[pl.ds(j * _MAXLEN, _MAXLEN)],
                    gsem[b],
                )
                for j in range(_SEQ_PER_CHUNK)
            ]

        pending = [None, None]
        pending[0] = stage(0, 0)
        for ci in range(nchunk):
            b = ci & 1
            nb = b ^ 1
            if ci + 1 < nchunk:
                if out_dma[nb] is not None:
                    out_dma[nb].wait()
                    out_dma[nb] = None
                pending[nb] = stage(ci + 1, nb)
            for cpy in pending[b]:
                cpy.wait()
            rv = rows[b]

            def lfn(l, carry, rv=rv):
                for d in range(_DSL):
                    sl = pl.ds(d * _LANES, _LANES)
                    p = pos_v[l, sl]
                    for s in range(_SEQ_PER_CHUNK):
                        r = l + s * _MAXLEN
                        rv[r, sl] = rv[r, sl] * _SCALE + p
                return carry

            lax.fori_loop(0, _MAXLEN, lfn, 0)
            goff = wid * per_w + ci * _C
            out_dma[b] = pltpu.async_copy(
                rv, out_hbm.at[pl.ds(goff, _C)], osem[b])

        for b in (0, 1):
            if out_dma[b] is not None:
                out_dma[b].wait()

    return body


def kernel(x, table):
    info = plsc.get_sparse_core_info()
    nw = info.num_cores * info.num_subcores
    nchunk = _ROWS // (nw * _C)
    pos = jnp.asarray(_pos_encoding())
    x32 = x.reshape(nw * nchunk, _SEQ_PER_CHUNK, _MAXLEN).astype(jnp.int32)

    mesh = plsc.VectorSubcoreMesh(core_axis_name="c", subcore_axis_name="s")
    kfn = pl.kernel(
        _make_body(nw, nchunk),
        mesh=mesh,
        compiler_params=pltpu.CompilerParams(use_tc_tiling_on_sc=False),
        out_type=jax.ShapeDtypeStruct((_ROWS, _EMBED), jnp.float32),
        scratch_types=[
            pltpu.VMEM((_SEQ_PER_CHUNK, _MAXLEN), jnp.int32),
            pltpu.VMEM((_SEQ_PER_CHUNK, _MAXLEN), jnp.int32),
            pltpu.VMEM((_C, _EMBED), jnp.float32),
            pltpu.VMEM((_C, _EMBED), jnp.float32),
            pltpu.VMEM((_MAXLEN, _EMBED), jnp.float32),
            pltpu.SemaphoreType.DMA,
            pltpu.SemaphoreType.DMA,
            pltpu.SemaphoreType.DMA,
            pltpu.SemaphoreType.DMA,
        ],
    )
    out = kfn(x32, pos, table)
    return out.reshape(_BATCH, _MAXLEN, _EMBED)
